# Initial kernel scaffold; baseline (speedup 1.0000x reference)
import jax
import jax.numpy as jnp
from jax.experimental import pallas as pl


def kernel(x, edge_index, edge_attr, W_enc1, b_enc1, W_enc2, b_enc2, Wm, bm, Wu, bu, W_dec1, b_dec1, W_dec2, b_dec2):
    raise NotImplementedError("write your pallas kernel here")


# speedup vs baseline: 1.7541x; 1.7541x over previous
import functools

import jax
import jax.numpy as jnp
from jax import lax
from jax.experimental import pallas as pl
from jax.experimental.pallas import tpu as pltpu
from jax.experimental.pallas import tpu_sc as plsc

_N = 10000
_E = 320000
_D = 128
_DE = 4
_L = 3
_OUT = 4

_NC = 2
_NS = 16
_NW = _NC * _NS
_EPW = _E // _NW
_CH = 80
_NCHUNK = _EPW // _CH
_NPAD = 10240
_RPS = _NPAD // _NS
_ZR = 128

_R = 1000
_GRID = _N // _R


def _rows(width):
    return pl.BlockSpec((_R, width), lambda i: (i, 0))


def _full(shape):
    return pl.BlockSpec(shape, lambda i: (0,) * len(shape))


def _dot(a, b):
    return jnp.dot(a, b, preferred_element_type=jnp.float32)


def _enc_body(x_ref, w1, b1, w2, b2, wms, wmd, h_ref, a_ref, b_ref):
    t = jnp.maximum(_dot(x_ref[...], w1[...]) + b1[...], 0.0)
    h = _dot(t, w2[...]) + b2[...]
    h_ref[...] = h
    a_ref[...] = _dot(h, wms[...])
    b_ref[...] = _dot(h, wmd[...])


def _upd_body(h_ref, p0, p1, wua, wub, bur, wms, wmd, h2_ref, a_ref, b_ref):
    agg = p0[...] + p1[...]
    h = h_ref[...]
    hn = jnp.maximum(_dot(h, wua[...]) + _dot(agg, wub[...]) + bur[...], 0.0)
    h2 = h + hn
    h2_ref[...] = h2
    a_ref[...] = _dot(h2, wms[...])
    b_ref[...] = _dot(h2, wmd[...])


def _fin_body(h_ref, p0, p1, wua, wub, bur, wd1, bd1, wd2, bd2, out_ref):
    agg = p0[...] + p1[...]
    h = h_ref[...]
    hn = jnp.maximum(_dot(h, wua[...]) + _dot(agg, wub[...]) + bur[...], 0.0)
    h2 = h + hn
    t = jnp.maximum(_dot(h2, wd1[...]) + bd1[...], 0.0)
    out_ref[...] = _dot(t, wd2[...]) + bd2[...]


_enc_call = pl.pallas_call(
    _enc_body,
    grid=(_GRID,),
    in_specs=[_rows(_D), _full((_D, _D)), _full((1, _D)), _full((_D, _D)),
              _full((1, _D)), _full((_D, _D)), _full((_D, _D))],
    out_specs=[_rows(_D), _rows(_D), _rows(_D)],
    out_shape=[jax.ShapeDtypeStruct((_N, _D), jnp.float32)] * 3,
)

_upd_call = pl.pallas_call(
    _upd_body,
    grid=(_GRID,),
    in_specs=[_rows(_D), _rows(_D), _rows(_D), _full((_D, _D)),
              _full((_D, _D)), _full((1, _D)), _full((_D, _D)), _full((_D, _D))],
    out_specs=[_rows(_D), _rows(_D), _rows(_D)],
    out_shape=[jax.ShapeDtypeStruct((_N, _D), jnp.float32)] * 3,
)

_fin_call = pl.pallas_call(
    _fin_body,
    grid=(_GRID,),
    in_specs=[_rows(_D), _rows(_D), _rows(_D), _full((_D, _D)),
              _full((_D, _D)), _full((1, _D)), _full((_D, _D)), _full((1, _D)),
              _full((_D, _OUT)), _full((1, _OUT))],
    out_specs=[_rows(_OUT)],
    out_shape=[jax.ShapeDtypeStruct((_N, _OUT), jnp.float32)],
)


@functools.cache
def _build_edge_pass():
    return pl.kernel(
        _edge_body,
        mesh=plsc.VectorSubcoreMesh(core_axis_name="c", subcore_axis_name="s"),
        out_type=jax.ShapeDtypeStruct((_NC, _NPAD, _D), jnp.float32),
    scratch_types=[
        pltpu.VMEM((_CH,), jnp.int32),
        pltpu.VMEM((_CH,), jnp.int32),
        pltpu.VMEM((_CH, _D), jnp.float32),
        pltpu.VMEM((_CH, _D), jnp.float32),
        pltpu.VMEM((_CH * _DE,), jnp.float32),
        pltpu.VMEM((_CH, _D), jnp.float32),
        pltpu.VMEM((_DE, _D), jnp.float32),
        pltpu.VMEM((_D,), jnp.float32),
        pltpu.VMEM((_ZR, _D), jnp.float32),
        pltpu.VMEM_SHARED((_NPAD, _D), jnp.float32),
        pltpu.SemaphoreType.DMA,
        pltpu.SemaphoreType.DMA,
    ],
    )


def _edge_body(a_hbm, b_hbm, src_hbm, dst_hbm, ea_hbm, wme_hbm, bm_hbm,
               out_hbm, srcv, dstv, arows, brows, eav, msgv, wmev, bmv,
               zbuf, aggsh, sem_a, sem_b):
    c = lax.axis_index("c")
    s = lax.axis_index("s")
    wid = s * _NC + c

    pltpu.sync_copy(wme_hbm, wmev)
    pltpu.sync_copy(bm_hbm, bmv)

    zero16 = jnp.zeros((16,), jnp.float32)

    def _zrow(r, carry):
        for j in range(_D // 16):
            zbuf[r, pl.ds(j * 16, 16)] = zero16
        return carry

    lax.fori_loop(0, _ZR, _zrow, 0)
    for k in range(_RPS // _ZR):
        pltpu.sync_copy(zbuf, aggsh.at[pl.ds(s * _RPS + k * _ZR, _ZR)])
    plsc.subcore_barrier()

    def _chunk(ci, carry):
        base = pl.multiple_of(wid * _EPW + ci * _CH, 8)
        pltpu.sync_copy(src_hbm.at[pl.ds(base, _CH)], srcv)
        pltpu.sync_copy(dst_hbm.at[pl.ds(base, _CH)], dstv)
        pltpu.sync_copy(ea_hbm.at[pl.ds(base * _DE, _CH * _DE)], eav)
        cp_a = pltpu.async_copy(a_hbm.at[srcv], arows, sem_a)
        cp_b = pltpu.async_copy(b_hbm.at[dstv], brows, sem_b)
        cp_a.wait()
        cp_b.wait()

        def _egroup(g, ecarry):
            w = eav[pl.ds(g * 16, 16)]
            for m in range(4):
                e = g * 4 + m
                ea0 = w[4 * m + 0]
                ea1 = w[4 * m + 1]
                ea2 = w[4 * m + 2]
                ea3 = w[4 * m + 3]
                for j in range(_D // 16):
                    sl = pl.ds(j * 16, 16)
                    v = arows[e, sl] + brows[e, sl] + bmv[sl]
                    v = v + ea0 * wmev[0, sl] + ea1 * wmev[1, sl]
                    v = v + ea2 * wmev[2, sl] + ea3 * wmev[3, sl]
                    msgv[e, sl] = jnp.maximum(v, 0.0)
            return ecarry

        lax.fori_loop(0, _CH // 4, _egroup, 0)
        pltpu.sync_copy(msgv, aggsh.at[dstv], add=True)
        return carry

    lax.fori_loop(0, _NCHUNK, _chunk, 0)
    plsc.subcore_barrier()

    for k in range(_RPS // _ZR):
        r0 = s * _RPS + k * _ZR
        pltpu.sync_copy(aggsh.at[pl.ds(r0, _ZR)], out_hbm.at[c, pl.ds(r0, _ZR)])


def kernel(x, edge_index, edge_attr, W_enc1, b_enc1, W_enc2, b_enc2,
           Wm, bm, Wu, bu, W_dec1, b_dec1, W_dec2, b_dec2):
    src = edge_index[0]
    dst = edge_index[1]
    ea_flat = edge_attr.reshape(_E * _DE)
    b_enc1r = b_enc1.reshape(1, _D)
    b_enc2r = b_enc2.reshape(1, _D)
    b_dec1r = b_dec1.reshape(1, _D)
    b_dec2r = b_dec2.reshape(1, _OUT)

    h, a_proj, b_proj = _enc_call(
        x, W_enc1, b_enc1r, W_enc2, b_enc2r, Wm[0, :_D], Wm[0, _D:2 * _D])

    preds = None
    for i in range(_L):
        parts = _build_edge_pass()(a_proj, b_proj, src, dst, ea_flat,
                                   Wm[i, 2 * _D:], bm[i])
        p0 = parts[0, :_N]
        p1 = parts[1, :_N]
        wua = Wu[i, :_D]
        wub = Wu[i, _D:]
        bur = bu[i].reshape(1, _D)
        if i < _L - 1:
            h, a_proj, b_proj = _upd_call(
                h, p0, p1, wua, wub, bur,
                Wm[i + 1, :_D], Wm[i + 1, _D:2 * _D])
        else:
            (preds,) = _fin_call(
                h, p0, p1, wua, wub, bur,
                W_dec1, b_dec1r, W_dec2, b_dec2r)
    return preds


# speedup vs baseline: 2.1907x; 1.2489x over previous
import functools

import jax
import jax.numpy as jnp
from jax import lax
from jax.experimental import pallas as pl
from jax.experimental.pallas import tpu as pltpu
from jax.experimental.pallas import tpu_sc as plsc

_N = 10000
_E = 320000
_D = 128
_DE = 4
_L = 3
_OUT = 4

_NC = 2
_NS = 16
_NW = _NC * _NS
_EPW = _E // _NW
_CH = 80
_NCHUNK = _EPW // _CH
_NPAD = 10240
_RPS = _NPAD // _NS

_R = 1000
_GRID = _N // _R


def _rows(width):
    return pl.BlockSpec((_R, width), lambda i: (i, 0))


def _full(shape):
    return pl.BlockSpec(shape, lambda i: (0,) * len(shape))


def _dot(a, b):
    return jnp.dot(a, b, preferred_element_type=jnp.float32)


def _enc_body(x_ref, w1, b1, w2, b2, wms, wmd, h_ref, a_ref, b_ref):
    t = jnp.maximum(_dot(x_ref[...], w1[...]) + b1[...], 0.0)
    h = _dot(t, w2[...]) + b2[...]
    h_ref[...] = h
    a_ref[...] = _dot(h, wms[...])
    b_ref[...] = _dot(h, wmd[...])


def _upd_body(h_ref, p0, p1, wua, wub, bur, wms, wmd, h2_ref, a_ref, b_ref):
    agg = p0[...] + p1[...]
    h = h_ref[...]
    hn = jnp.maximum(_dot(h, wua[...]) + _dot(agg, wub[...]) + bur[...], 0.0)
    h2 = h + hn
    h2_ref[...] = h2
    a_ref[...] = _dot(h2, wms[...])
    b_ref[...] = _dot(h2, wmd[...])


def _fin_body(h_ref, p0, p1, wua, wub, bur, wd1, bd1, wd2, bd2, out_ref):
    agg = p0[...] + p1[...]
    h = h_ref[...]
    hn = jnp.maximum(_dot(h, wua[...]) + _dot(agg, wub[...]) + bur[...], 0.0)
    h2 = h + hn
    t = jnp.maximum(_dot(h2, wd1[...]) + bd1[...], 0.0)
    out_ref[...] = _dot(t, wd2[...]) + bd2[...]


_enc_call = pl.pallas_call(
    _enc_body,
    grid=(_GRID,),
    in_specs=[_rows(_D), _full((_D, _D)), _full((1, _D)), _full((_D, _D)),
              _full((1, _D)), _full((_D, _D)), _full((_D, _D))],
    out_specs=[_rows(_D), _rows(_D), _rows(_D)],
    out_shape=[jax.ShapeDtypeStruct((_N, _D), jnp.float32)] * 3,
)

_upd_call = pl.pallas_call(
    _upd_body,
    grid=(_GRID,),
    in_specs=[_rows(_D), _rows(_D), _rows(_D), _full((_D, _D)),
              _full((_D, _D)), _full((1, _D)), _full((_D, _D)), _full((_D, _D))],
    out_specs=[_rows(_D), _rows(_D), _rows(_D)],
    out_shape=[jax.ShapeDtypeStruct((_N, _D), jnp.float32)] * 3,
)

_fin_call = pl.pallas_call(
    _fin_body,
    grid=(_GRID,),
    in_specs=[_rows(_D), _rows(_D), _rows(_D), _full((_D, _D)),
              _full((_D, _D)), _full((1, _D)), _full((_D, _D)), _full((1, _D)),
              _full((_D, _OUT)), _full((1, _OUT))],
    out_specs=[_rows(_OUT)],
    out_shape=[jax.ShapeDtypeStruct((_N, _OUT), jnp.float32)],
)


@functools.cache
def _build_edge_pass():
    return pl.kernel(
        _edge_body,
        mesh=plsc.VectorSubcoreMesh(core_axis_name="c", subcore_axis_name="s"),
        out_type=jax.ShapeDtypeStruct((_NC, _NPAD, _D), jnp.float32),
    scratch_types=[
        pltpu.VMEM((_CH,), jnp.int32),
        pltpu.VMEM((_CH,), jnp.int32),
        pltpu.VMEM((_CH * _DE,), jnp.float32),
        pltpu.VMEM((_CH,), jnp.int32),
        pltpu.VMEM((_CH,), jnp.int32),
        pltpu.VMEM((_CH * _DE,), jnp.float32),
        pltpu.VMEM((_CH, _D), jnp.float32),
        pltpu.VMEM((_CH, _D), jnp.float32),
        pltpu.VMEM((_CH, _D), jnp.float32),
        pltpu.VMEM((_CH, _D), jnp.float32),
        pltpu.VMEM((_DE, _D), jnp.float32),
        pltpu.VMEM((_D,), jnp.float32),
        pltpu.VMEM_SHARED((_NPAD, _D), jnp.float32),
        pltpu.SemaphoreType.DMA,
        pltpu.SemaphoreType.DMA,
        pltpu.SemaphoreType.DMA,
        pltpu.SemaphoreType.DMA,
        pltpu.SemaphoreType.DMA,
        pltpu.SemaphoreType.DMA,
    ],
    )


def _edge_body(a_hbm, b_hbm, src_hbm, dst_hbm, ea_hbm, wme_hbm, bm_hbm,
               z_hbm, out_hbm, src_a, dst_a, ea_a, src_b, dst_b, ea_b,
               arows_a, brows_a, arows_b, brows_b, wmev, bmv,
               aggsh, semi_a, semi_b, sga_a, sgb_a, sga_b, sgb_b):
    c = lax.axis_index("c")
    s = lax.axis_index("s")
    wid = s * _NC + c

    bufs = (
        (src_a, dst_a, ea_a, arows_a, brows_a, semi_a, sga_a, sgb_a),
        (src_b, dst_b, ea_b, arows_b, brows_b, semi_b, sga_b, sgb_b),
    )

    def issue_idx(ci, p):
        srcv, dstv, eav, _, _, semi, _, _ = bufs[p]
        base = pl.multiple_of(wid * _EPW + ci * _CH, 8)
        pltpu.async_copy(src_hbm.at[pl.ds(base, _CH)], srcv, semi)
        pltpu.async_copy(dst_hbm.at[pl.ds(base, _CH)], dstv, semi)
        pltpu.async_copy(ea_hbm.at[pl.ds(base * _DE, _CH * _DE)], eav, semi)

    def wait_idx(p):
        srcv, dstv, eav, _, _, semi, _, _ = bufs[p]
        pltpu.make_async_copy(src_hbm.at[pl.ds(0, _CH)], srcv, semi).wait()
        pltpu.make_async_copy(dst_hbm.at[pl.ds(0, _CH)], dstv, semi).wait()
        pltpu.make_async_copy(
            ea_hbm.at[pl.ds(0, _CH * _DE)], eav, semi).wait()

    def issue_gather(p):
        srcv, dstv, _, arows, brows, _, sga, sgb = bufs[p]
        pltpu.async_copy(a_hbm.at[srcv], arows, sga)
        pltpu.async_copy(b_hbm.at[dstv], brows, sgb)

    def wait_gather(p):
        srcv, dstv, _, arows, brows, _, sga, sgb = bufs[p]
        pltpu.make_async_copy(a_hbm.at[srcv], arows, sga).wait()
        pltpu.make_async_copy(b_hbm.at[dstv], brows, sgb).wait()

    def compute_scatter(p):
        _, dstv, eav, arows, brows, _, _, _ = bufs[p]

        def _egroup(g, ecarry):
            w = eav[pl.ds(g * 16, 16)]
            for m in range(4):
                e = g * 4 + m
                ea0 = w[4 * m + 0]
                ea1 = w[4 * m + 1]
                ea2 = w[4 * m + 2]
                ea3 = w[4 * m + 3]
                for j in range(_D // 16):
                    sl = pl.ds(j * 16, 16)
                    v = arows[e, sl] + brows[e, sl] + bmv[sl]
                    v = v + ea0 * wmev[0, sl] + ea1 * wmev[1, sl]
                    v = v + ea2 * wmev[2, sl] + ea3 * wmev[3, sl]
                    arows[e, sl] = jnp.maximum(v, 0.0)
            return ecarry

        lax.fori_loop(0, _CH // 4, _egroup, 0)
        pltpu.sync_copy(arows, aggsh.at[dstv], add=True)

    issue_idx(0, 0)
    issue_idx(1, 1)

    pltpu.sync_copy(wme_hbm, wmev)
    pltpu.sync_copy(bm_hbm, bmv)

    pltpu.sync_copy(z_hbm, aggsh.at[pl.ds(s * _RPS, _RPS)])
    plsc.subcore_barrier()

    wait_idx(0)
    issue_gather(0)

    def _pair(k2, carry):
        c0 = k2 * 2
        wait_gather(0)
        wait_idx(1)
        issue_gather(1)
        compute_scatter(0)
        issue_idx(c0 + 2, 0)
        wait_gather(1)
        wait_idx(0)
        issue_gather(0)
        compute_scatter(1)

        @pl.when(c0 + 3 < _NCHUNK)
        def _():
            issue_idx(c0 + 3, 1)

        return carry

    lax.fori_loop(0, (_NCHUNK - 1) // 2, _pair, 0)

    wait_gather(0)
    compute_scatter(0)
    plsc.subcore_barrier()

    r0 = s * _RPS
    pltpu.sync_copy(aggsh.at[pl.ds(r0, _RPS)], out_hbm.at[c, pl.ds(r0, _RPS)])


def kernel(x, edge_index, edge_attr, W_enc1, b_enc1, W_enc2, b_enc2,
           Wm, bm, Wu, bu, W_dec1, b_dec1, W_dec2, b_dec2):
    src = edge_index[0]
    dst = edge_index[1]
    ea_flat = edge_attr.reshape(_E * _DE)
    zeros_blk = jnp.zeros((_RPS, _D), jnp.float32)
    b_enc1r = b_enc1.reshape(1, _D)
    b_enc2r = b_enc2.reshape(1, _D)
    b_dec1r = b_dec1.reshape(1, _D)
    b_dec2r = b_dec2.reshape(1, _OUT)

    h, a_proj, b_proj = _enc_call(
        x, W_enc1, b_enc1r, W_enc2, b_enc2r, Wm[0, :_D], Wm[0, _D:2 * _D])

    preds = None
    for i in range(_L):
        parts = _build_edge_pass()(a_proj, b_proj, src, dst, ea_flat,
                                   Wm[i, 2 * _D:], bm[i], zeros_blk)
        p0 = parts[0, :_N]
        p1 = parts[1, :_N]
        wua = Wu[i, :_D]
        wub = Wu[i, _D:]
        bur = bu[i].reshape(1, _D)
        if i < _L - 1:
            h, a_proj, b_proj = _upd_call(
                h, p0, p1, wua, wub, bur,
                Wm[i + 1, :_D], Wm[i + 1, _D:2 * _D])
        else:
            (preds,) = _fin_call(
                h, p0, p1, wua, wub, bur,
                W_dec1, b_dec1r, W_dec2, b_dec2r)
    return preds


# speedup vs baseline: 2.4419x; 1.1146x over previous
import functools

import jax
import jax.numpy as jnp
from jax import lax
from jax.experimental import pallas as pl
from jax.experimental.pallas import tpu as pltpu
from jax.experimental.pallas import tpu_sc as plsc

_N = 10000
_E = 320000
_D = 128
_DE = 4
_L = 3
_OUT = 4

_NC = 2
_NS = 16
_NW = _NC * _NS
_EPW = _E // _NW
_CH = 80
_NCHUNK = _EPW // _CH
_NPAD = 10240
_RPS = _NPAD // _NS

_R = 1000
_GRID = _N // _R


def _rows(width):
    return pl.BlockSpec((_R, width), lambda i: (i, 0))


def _full(shape):
    return pl.BlockSpec(shape, lambda i: (0,) * len(shape))


def _dot(a, b):
    return jnp.dot(a, b, preferred_element_type=jnp.float32)


def _enc_body(x_ref, w1, b1, w2, b2, wms, wmd, bmr, h_ref, a_ref, b_ref):
    t = jnp.maximum(_dot(x_ref[...], w1[...]) + b1[...], 0.0)
    h = _dot(t, w2[...]) + b2[...]
    h_ref[...] = h
    a_ref[...] = _dot(h, wms[...])
    b_ref[...] = _dot(h, wmd[...]) + bmr[...]


def _upd_body(h_ref, p0, p1, wua, wub, bur, wms, wmd, bmr, h2_ref, a_ref,
              b_ref):
    agg = p0[...] + p1[...]
    h = h_ref[...]
    hn = jnp.maximum(_dot(h, wua[...]) + _dot(agg, wub[...]) + bur[...], 0.0)
    h2 = h + hn
    h2_ref[...] = h2
    a_ref[...] = _dot(h2, wms[...])
    b_ref[...] = _dot(h2, wmd[...]) + bmr[...]


def _fin_body(h_ref, p0, p1, wua, wub, bur, wd1, bd1, wd2, bd2, out_ref):
    agg = p0[...] + p1[...]
    h = h_ref[...]
    hn = jnp.maximum(_dot(h, wua[...]) + _dot(agg, wub[...]) + bur[...], 0.0)
    h2 = h + hn
    t = jnp.maximum(_dot(h2, wd1[...]) + bd1[...], 0.0)
    out_ref[...] = _dot(t, wd2[...]) + bd2[...]


_enc_call = pl.pallas_call(
    _enc_body,
    grid=(_GRID,),
    in_specs=[_rows(_D), _full((_D, _D)), _full((1, _D)), _full((_D, _D)),
              _full((1, _D)), _full((_D, _D)), _full((_D, _D)),
              _full((1, _D))],
    out_specs=[_rows(_D), _rows(_D), _rows(_D)],
    out_shape=[jax.ShapeDtypeStruct((_N, _D), jnp.float32)] * 3,
)

_upd_call = pl.pallas_call(
    _upd_body,
    grid=(_GRID,),
    in_specs=[_rows(_D), _rows(_D), _rows(_D), _full((_D, _D)),
              _full((_D, _D)), _full((1, _D)), _full((_D, _D)),
              _full((_D, _D)), _full((1, _D))],
    out_specs=[_rows(_D), _rows(_D), _rows(_D)],
    out_shape=[jax.ShapeDtypeStruct((_N, _D), jnp.float32)] * 3,
)

_fin_call = pl.pallas_call(
    _fin_body,
    grid=(_GRID,),
    in_specs=[_rows(_D), _rows(_D), _rows(_D), _full((_D, _D)),
              _full((_D, _D)), _full((1, _D)), _full((_D, _D)), _full((1, _D)),
              _full((_D, _OUT)), _full((1, _OUT))],
    out_specs=[_rows(_OUT)],
    out_shape=[jax.ShapeDtypeStruct((_N, _OUT), jnp.float32)],
)


@functools.cache
def _build_edge_pass():
    return pl.kernel(
        _edge_body,
        mesh=plsc.VectorSubcoreMesh(core_axis_name="c", subcore_axis_name="s"),
        out_type=jax.ShapeDtypeStruct((_NC, _NPAD, _D), jnp.float32),
    scratch_types=[
        pltpu.VMEM((_CH,), jnp.int32),
        pltpu.VMEM((_CH,), jnp.int32),
        pltpu.VMEM((_CH * _DE,), jnp.float32),
        pltpu.VMEM((_CH,), jnp.int32),
        pltpu.VMEM((_CH,), jnp.int32),
        pltpu.VMEM((_CH * _DE,), jnp.float32),
        pltpu.VMEM((_CH, _D), jnp.float32),
        pltpu.VMEM((_CH, _D), jnp.float32),
        pltpu.VMEM((_CH, _D), jnp.float32),
        pltpu.VMEM((_CH, _D), jnp.float32),
        pltpu.VMEM((_DE, _D), jnp.float32),
        pltpu.VMEM_SHARED((_NPAD, _D), jnp.float32),
        pltpu.SemaphoreType.DMA,
        pltpu.SemaphoreType.DMA,
        pltpu.SemaphoreType.DMA,
        pltpu.SemaphoreType.DMA,
        pltpu.SemaphoreType.DMA,
        pltpu.SemaphoreType.DMA,
    ],
    )


def _edge_body(a_hbm, b_hbm, src_hbm, dst_hbm, ea_hbm, wme_hbm,
               z_hbm, out_hbm, src_a, dst_a, ea_a, src_b, dst_b, ea_b,
               arows_a, brows_a, arows_b, brows_b, wmev,
               aggsh, semi_a, semi_b, sga_a, sgb_a, sga_b, sgb_b):
    c = lax.axis_index("c")
    s = lax.axis_index("s")
    wid = s * _NC + c

    bufs = (
        (src_a, dst_a, ea_a, arows_a, brows_a, semi_a, sga_a, sgb_a),
        (src_b, dst_b, ea_b, arows_b, brows_b, semi_b, sga_b, sgb_b),
    )

    def issue_idx(ci, p):
        srcv, dstv, eav, _, _, semi, _, _ = bufs[p]
        base = pl.multiple_of(wid * _EPW + ci * _CH, 8)
        pltpu.async_copy(src_hbm.at[pl.ds(base, _CH)], srcv, semi)
        pltpu.async_copy(dst_hbm.at[pl.ds(base, _CH)], dstv, semi)
        pltpu.async_copy(ea_hbm.at[pl.ds(base * _DE, _CH * _DE)], eav, semi)

    def wait_idx(p):
        srcv, dstv, eav, _, _, semi, _, _ = bufs[p]
        pltpu.make_async_copy(src_hbm.at[pl.ds(0, _CH)], srcv, semi).wait()
        pltpu.make_async_copy(dst_hbm.at[pl.ds(0, _CH)], dstv, semi).wait()
        pltpu.make_async_copy(
            ea_hbm.at[pl.ds(0, _CH * _DE)], eav, semi).wait()

    def issue_gather(p):
        srcv, dstv, _, arows, brows, _, sga, sgb = bufs[p]
        pltpu.async_copy(a_hbm.at[srcv], arows, sga)
        pltpu.async_copy(b_hbm.at[dstv], brows, sgb)

    def wait_gather(p):
        srcv, dstv, _, arows, brows, _, sga, sgb = bufs[p]
        pltpu.make_async_copy(a_hbm.at[srcv], arows, sga).wait()
        pltpu.make_async_copy(b_hbm.at[dstv], brows, sgb).wait()

    def compute_scatter(p):
        _, dstv, eav, arows, brows, _, _, _ = bufs[p]

        @plsc.parallel_loop(0, _CH // 4, unroll=2)
        def _egroup(g):
            w = eav[pl.ds(g * 16, 16)]
            for m in range(4):
                e = g * 4 + m
                ea0 = w[4 * m + 0]
                ea1 = w[4 * m + 1]
                ea2 = w[4 * m + 2]
                ea3 = w[4 * m + 3]
                for j in range(_D // 16):
                    sl = pl.ds(j * 16, 16)
                    ab = arows[e, sl] + brows[e, sl]
                    p01 = ea0 * wmev[0, sl] + ea1 * wmev[1, sl]
                    p23 = ea2 * wmev[2, sl] + ea3 * wmev[3, sl]
                    v = ab + (p01 + p23)
                    arows[e, sl] = jnp.maximum(v, 0.0)

        pltpu.sync_copy(arows, aggsh.at[dstv], add=True)

    issue_idx(0, 0)
    issue_idx(1, 1)

    pltpu.sync_copy(wme_hbm, wmev)

    pltpu.sync_copy(z_hbm, aggsh.at[pl.ds(s * _RPS, _RPS)])
    plsc.subcore_barrier()

    wait_idx(0)
    issue_gather(0)

    def _pair(k2, carry):
        c0 = k2 * 2
        wait_gather(0)
        wait_idx(1)
        issue_gather(1)
        compute_scatter(0)
        issue_idx(c0 + 2, 0)
        wait_gather(1)
        wait_idx(0)
        issue_gather(0)
        compute_scatter(1)

        @pl.when(c0 + 3 < _NCHUNK)
        def _():
            issue_idx(c0 + 3, 1)

        return carry

    lax.fori_loop(0, (_NCHUNK - 1) // 2, _pair, 0)

    wait_gather(0)
    compute_scatter(0)
    plsc.subcore_barrier()

    r0 = s * _RPS
    pltpu.sync_copy(aggsh.at[pl.ds(r0, _RPS)], out_hbm.at[c, pl.ds(r0, _RPS)])


def kernel(x, edge_index, edge_attr, W_enc1, b_enc1, W_enc2, b_enc2,
           Wm, bm, Wu, bu, W_dec1, b_dec1, W_dec2, b_dec2):
    src = edge_index[0]
    dst = edge_index[1]
    ea_flat = edge_attr.reshape(_E * _DE)
    zeros_blk = jnp.zeros((_RPS, _D), jnp.float32)
    b_enc1r = b_enc1.reshape(1, _D)
    b_enc2r = b_enc2.reshape(1, _D)
    b_dec1r = b_dec1.reshape(1, _D)
    b_dec2r = b_dec2.reshape(1, _OUT)

    h, a_proj, b_proj = _enc_call(
        x, W_enc1, b_enc1r, W_enc2, b_enc2r, Wm[0, :_D], Wm[0, _D:2 * _D],
        bm[0].reshape(1, _D))

    preds = None
    for i in range(_L):
        parts = _build_edge_pass()(a_proj, b_proj, src, dst, ea_flat,
                                   Wm[i, 2 * _D:], zeros_blk)
        p0 = parts[0, :_N]
        p1 = parts[1, :_N]
        wua = Wu[i, :_D]
        wub = Wu[i, _D:]
        bur = bu[i].reshape(1, _D)
        if i < _L - 1:
            h, a_proj, b_proj = _upd_call(
                h, p0, p1, wua, wub, bur,
                Wm[i + 1, :_D], Wm[i + 1, _D:2 * _D],
                bm[i + 1].reshape(1, _D))
        else:
            (preds,) = _fin_call(
                h, p0, p1, wua, wub, bur,
                W_dec1, b_dec1r, W_dec2, b_dec2r)
    return preds


# speedup vs baseline: 3.5935x; 1.4716x over previous
import functools

import jax
import jax.numpy as jnp
from jax import lax
from jax.experimental import pallas as pl
from jax.experimental.pallas import tpu as pltpu
from jax.experimental.pallas import tpu_sc as plsc

_N = 10000
_E = 320000
_D = 128
_DE = 4
_L = 3
_OUT = 4

_NC = 2
_NS = 16
_NW = _NC * _NS
_EPW = _E // _NW
_CH = 80
_NCHUNK = _EPW // _CH
_NPAD = 10240
_RPS = _NPAD // _NS

_R = 1000
_GRID = _N // _R


def _rows(width):
    return pl.BlockSpec((_R, width), lambda i: (i, 0))


def _full(shape):
    return pl.BlockSpec(shape, lambda i: (0,) * len(shape))


def _dot(a, b):
    return jnp.dot(a, b, preferred_element_type=jnp.float32)


def _enc_body(x_ref, w1, b1, w2, b2, wms, wmd, bmr, h_ref, a_ref, b_ref):
    t = jnp.maximum(_dot(x_ref[...], w1[...]) + b1[...], 0.0)
    h = _dot(t, w2[...]) + b2[...]
    h_ref[...] = h
    a_ref[...] = _dot(h, wms[...])
    b_ref[...] = _dot(h, wmd[...]) + bmr[...]


def _upd_body(h_ref, p0, p1, wua, wub, bur, wms, wmd, bmr, h2_ref, a_ref,
              b_ref):
    agg = p0[...] + p1[...]
    h = h_ref[...]
    hn = jnp.maximum(_dot(h, wua[...]) + _dot(agg, wub[...]) + bur[...], 0.0)
    h2 = h + hn
    h2_ref[...] = h2
    a_ref[...] = _dot(h2, wms[...])
    b_ref[...] = _dot(h2, wmd[...]) + bmr[...]


def _fin_body(h_ref, p0, p1, wua, wub, bur, wd1, bd1, wd2, bd2, out_ref):
    agg = p0[...] + p1[...]
    h = h_ref[...]
    hn = jnp.maximum(_dot(h, wua[...]) + _dot(agg, wub[...]) + bur[...], 0.0)
    h2 = h + hn
    t = jnp.maximum(_dot(h2, wd1[...]) + bd1[...], 0.0)
    out_ref[...] = _dot(t, wd2[...]) + bd2[...]


_enc_call = pl.pallas_call(
    _enc_body,
    grid=(_GRID,),
    in_specs=[_rows(_D), _full((_D, _D)), _full((1, _D)), _full((_D, _D)),
              _full((1, _D)), _full((_D, _D)), _full((_D, _D)),
              _full((1, _D))],
    out_specs=[_rows(_D), _rows(_D), _rows(_D)],
    out_shape=[jax.ShapeDtypeStruct((_N, _D), jnp.float32)] * 3,
)

_upd_call = pl.pallas_call(
    _upd_body,
    grid=(_GRID,),
    in_specs=[_rows(_D), _rows(_D), _rows(_D), _full((_D, _D)),
              _full((_D, _D)), _full((1, _D)), _full((_D, _D)),
              _full((_D, _D)), _full((1, _D))],
    out_specs=[_rows(_D), _rows(_D), _rows(_D)],
    out_shape=[jax.ShapeDtypeStruct((_N, _D), jnp.float32)] * 3,
)

_fin_call = pl.pallas_call(
    _fin_body,
    grid=(_GRID,),
    in_specs=[_rows(_D), _rows(_D), _rows(_D), _full((_D, _D)),
              _full((_D, _D)), _full((1, _D)), _full((_D, _D)), _full((1, _D)),
              _full((_D, _OUT)), _full((1, _OUT))],
    out_specs=[_rows(_OUT)],
    out_shape=[jax.ShapeDtypeStruct((_N, _OUT), jnp.float32)],
)


@functools.cache
def _build_edge_pass():
    return pl.kernel(
        _edge_body,
        mesh=plsc.VectorSubcoreMesh(core_axis_name="c", subcore_axis_name="s"),
        out_type=jax.ShapeDtypeStruct((_NC, _NPAD, _D), jnp.float32),
    scratch_types=[
        pltpu.VMEM((_CH,), jnp.int32),
        pltpu.VMEM((_CH,), jnp.int32),
        pltpu.VMEM((_CH * _DE,), jnp.float32),
        pltpu.VMEM((_CH,), jnp.int32),
        pltpu.VMEM((_CH,), jnp.int32),
        pltpu.VMEM((_CH * _DE,), jnp.float32),
        pltpu.VMEM((_CH, _D), jnp.float32),
        pltpu.VMEM((_CH, _D), jnp.float32),
        pltpu.VMEM((_CH, _D), jnp.float32),
        pltpu.VMEM((_CH, _D), jnp.float32),
        pltpu.VMEM((_DE, _D), jnp.float32),
        pltpu.VMEM_SHARED((_NPAD, _D), jnp.float32),
        pltpu.SemaphoreType.DMA,
        pltpu.SemaphoreType.DMA,
        pltpu.SemaphoreType.DMA,
        pltpu.SemaphoreType.DMA,
        pltpu.SemaphoreType.DMA,
        pltpu.SemaphoreType.DMA,
    ],
    )


def _edge_body(a_hbm, b_hbm, src_hbm, dst_hbm, ea_hbm, wme_hbm,
               z_hbm, out_hbm, src_a, dst_a, ea_a, src_b, dst_b, ea_b,
               arows_a, brows_a, arows_b, brows_b, wmev,
               aggsh, semi_a, semi_b, sga_a, sgb_a, sga_b, sgb_b):
    c = lax.axis_index("c")
    s = lax.axis_index("s")
    wid = s * _NC + c

    bufs = (
        (src_a, dst_a, ea_a, arows_a, brows_a, semi_a, sga_a, sgb_a),
        (src_b, dst_b, ea_b, arows_b, brows_b, semi_b, sga_b, sgb_b),
    )

    def issue_idx(ci, p):
        srcv, dstv, eav, _, _, semi, _, _ = bufs[p]
        base = pl.multiple_of(wid * _EPW + ci * _CH, 8)
        pltpu.async_copy(src_hbm.at[pl.ds(base, _CH)], srcv, semi)
        pltpu.async_copy(dst_hbm.at[pl.ds(base, _CH)], dstv, semi)
        pltpu.async_copy(ea_hbm.at[pl.ds(base * _DE, _CH * _DE)], eav, semi)

    def wait_idx(p):
        srcv, dstv, eav, _, _, semi, _, _ = bufs[p]
        pltpu.make_async_copy(src_hbm.at[pl.ds(0, _CH)], srcv, semi).wait()
        pltpu.make_async_copy(dst_hbm.at[pl.ds(0, _CH)], dstv, semi).wait()
        pltpu.make_async_copy(
            ea_hbm.at[pl.ds(0, _CH * _DE)], eav, semi).wait()

    def issue_gather(p):
        srcv, dstv, _, arows, brows, _, sga, sgb = bufs[p]
        pltpu.async_copy(a_hbm.at[srcv], arows, sga)
        pltpu.async_copy(b_hbm.at[dstv], brows, sgb)

    def wait_gather(p):
        srcv, dstv, _, arows, brows, _, sga, sgb = bufs[p]
        pltpu.make_async_copy(a_hbm.at[srcv], arows, sga).wait()
        pltpu.make_async_copy(b_hbm.at[dstv], brows, sgb).wait()

    def compute_scatter(p):
        _, dstv, eav, arows, brows, _, _, _ = bufs[p]

        for half in range(2):
            jlo = half * (_D // 32)

            wregs = [[wmev[k, pl.ds((jlo + j) * 16, 16)]
                      for j in range(_D // 32)] for k in range(_DE)]

            @plsc.parallel_loop(0, _CH // 4, unroll=2)
            def _egroup(g, wregs=wregs, jlo=jlo):
                w = eav[pl.ds(g * 16, 16)]
                for m in range(4):
                    e = g * 4 + m
                    ea0 = w[4 * m + 0]
                    ea1 = w[4 * m + 1]
                    ea2 = w[4 * m + 2]
                    ea3 = w[4 * m + 3]
                    for j in range(_D // 32):
                        sl = pl.ds((jlo + j) * 16, 16)
                        ab = arows[e, sl] + brows[e, sl]
                        p01 = ea0 * wregs[0][j] + ea1 * wregs[1][j]
                        p23 = ea2 * wregs[2][j] + ea3 * wregs[3][j]
                        v = ab + (p01 + p23)
                        arows[e, sl] = jnp.maximum(v, 0.0)

        pltpu.sync_copy(arows, aggsh.at[dstv], add=True)

    issue_idx(0, 0)
    issue_idx(1, 1)

    pltpu.sync_copy(wme_hbm, wmev)

    pltpu.sync_copy(z_hbm, aggsh.at[pl.ds(s * _RPS, _RPS)])
    plsc.subcore_barrier()

    wait_idx(0)
    issue_gather(0)

    def _pair(k2, carry):
        c0 = k2 * 2
        wait_gather(0)
        wait_idx(1)
        issue_gather(1)
        compute_scatter(0)
        issue_idx(c0 + 2, 0)
        wait_gather(1)
        wait_idx(0)
        issue_gather(0)
        compute_scatter(1)

        @pl.when(c0 + 3 < _NCHUNK)
        def _():
            issue_idx(c0 + 3, 1)

        return carry

    lax.fori_loop(0, (_NCHUNK - 1) // 2, _pair, 0)

    wait_gather(0)
    compute_scatter(0)
    plsc.subcore_barrier()

    r0 = s * _RPS
    pltpu.sync_copy(aggsh.at[pl.ds(r0, _RPS)], out_hbm.at[c, pl.ds(r0, _RPS)])


def kernel(x, edge_index, edge_attr, W_enc1, b_enc1, W_enc2, b_enc2,
           Wm, bm, Wu, bu, W_dec1, b_dec1, W_dec2, b_dec2):
    src = edge_index[0]
    dst = edge_index[1]
    ea_flat = edge_attr.reshape(_E * _DE)
    zeros_blk = jnp.zeros((_RPS, _D), jnp.float32)
    b_enc1r = b_enc1.reshape(1, _D)
    b_enc2r = b_enc2.reshape(1, _D)
    b_dec1r = b_dec1.reshape(1, _D)
    b_dec2r = b_dec2.reshape(1, _OUT)

    h, a_proj, b_proj = _enc_call(
        x, W_enc1, b_enc1r, W_enc2, b_enc2r, Wm[0, :_D], Wm[0, _D:2 * _D],
        bm[0].reshape(1, _D))

    preds = None
    for i in range(_L):
        parts = _build_edge_pass()(a_proj, b_proj, src, dst, ea_flat,
                                   Wm[i, 2 * _D:], zeros_blk)
        p0 = parts[0, :_N]
        p1 = parts[1, :_N]
        wua = Wu[i, :_D]
        wub = Wu[i, _D:]
        bur = bu[i].reshape(1, _D)
        if i < _L - 1:
            h, a_proj, b_proj = _upd_call(
                h, p0, p1, wua, wub, bur,
                Wm[i + 1, :_D], Wm[i + 1, _D:2 * _D],
                bm[i + 1].reshape(1, _D))
        else:
            (preds,) = _fin_call(
                h, p0, p1, wua, wub, bur,
                W_dec1, b_dec1r, W_dec2, b_dec2r)
    return preds


# speedup vs baseline: 4.3488x; 1.2102x over previous
import functools

import jax
import jax.numpy as jnp
from jax import lax
from jax.experimental import pallas as pl
from jax.experimental.pallas import tpu as pltpu
from jax.experimental.pallas import tpu_sc as plsc

_N = 10000
_E = 320000
_D = 128
_DE = 4
_L = 3
_OUT = 4

_NC = 2
_NS = 16
_NW = _NC * _NS
_EPW = _E // _NW
_CH = 80
_NCHUNK = _EPW // _CH
_NPAD = 10240
_RPS = _NPAD // _NS

_NPASS = 4
_UNROLL = 2

_R = 1000
_GRID = _N // _R


def _rows(width):
    return pl.BlockSpec((_R, width), lambda i: (i, 0))


def _full(shape):
    return pl.BlockSpec(shape, lambda i: (0,) * len(shape))


def _dot(a, b):
    return jnp.dot(a, b, preferred_element_type=jnp.float32)


def _enc_body(x_ref, w1, b1, w2, b2, wms, wmd, bmr, h_ref, a_ref, b_ref):
    t = jnp.maximum(_dot(x_ref[...], w1[...]) + b1[...], 0.0)
    h = _dot(t, w2[...]) + b2[...]
    h_ref[...] = h
    a_ref[...] = _dot(h, wms[...])
    b_ref[...] = _dot(h, wmd[...]) + bmr[...]


def _upd_body(h_ref, p0, p1, wua, wub, bur, wms, wmd, bmr, h2_ref, a_ref,
              b_ref):
    agg = p0[...] + p1[...]
    h = h_ref[...]
    hn = jnp.maximum(_dot(h, wua[...]) + _dot(agg, wub[...]) + bur[...], 0.0)
    h2 = h + hn
    h2_ref[...] = h2
    a_ref[...] = _dot(h2, wms[...])
    b_ref[...] = _dot(h2, wmd[...]) + bmr[...]


def _fin_body(h_ref, p0, p1, wua, wub, bur, wd1, bd1, wd2, bd2, out_ref):
    agg = p0[...] + p1[...]
    h = h_ref[...]
    hn = jnp.maximum(_dot(h, wua[...]) + _dot(agg, wub[...]) + bur[...], 0.0)
    h2 = h + hn
    t = jnp.maximum(_dot(h2, wd1[...]) + bd1[...], 0.0)
    out_ref[...] = _dot(t, wd2[...]) + bd2[...]


_enc_call = pl.pallas_call(
    _enc_body,
    grid=(_GRID,),
    in_specs=[_rows(_D), _full((_D, _D)), _full((1, _D)), _full((_D, _D)),
              _full((1, _D)), _full((_D, _D)), _full((_D, _D)),
              _full((1, _D))],
    out_specs=[_rows(_D), _rows(_D), _rows(_D)],
    out_shape=[jax.ShapeDtypeStruct((_N, _D), jnp.float32)] * 3,
)

_upd_call = pl.pallas_call(
    _upd_body,
    grid=(_GRID,),
    in_specs=[_rows(_D), _rows(_D), _rows(_D), _full((_D, _D)),
              _full((_D, _D)), _full((1, _D)), _full((_D, _D)),
              _full((_D, _D)), _full((1, _D))],
    out_specs=[_rows(_D), _rows(_D), _rows(_D)],
    out_shape=[jax.ShapeDtypeStruct((_N, _D), jnp.float32)] * 3,
)

_fin_call = pl.pallas_call(
    _fin_body,
    grid=(_GRID,),
    in_specs=[_rows(_D), _rows(_D), _rows(_D), _full((_D, _D)),
              _full((_D, _D)), _full((1, _D)), _full((_D, _D)), _full((1, _D)),
              _full((_D, _OUT)), _full((1, _OUT))],
    out_specs=[_rows(_OUT)],
    out_shape=[jax.ShapeDtypeStruct((_N, _OUT), jnp.float32)],
)


@functools.cache
def _build_edge_pass():
    return pl.kernel(
        _edge_body,
        mesh=plsc.VectorSubcoreMesh(core_axis_name="c", subcore_axis_name="s"),
        out_type=jax.ShapeDtypeStruct((_NC, _NPAD, _D), jnp.float32),
    scratch_types=[
        pltpu.VMEM((_CH,), jnp.int32),
        pltpu.VMEM((_CH,), jnp.int32),
        pltpu.VMEM((_CH * _DE,), jnp.float32),
        pltpu.VMEM((_CH,), jnp.int32),
        pltpu.VMEM((_CH,), jnp.int32),
        pltpu.VMEM((_CH * _DE,), jnp.float32),
        pltpu.VMEM((_CH, _D), jnp.float32),
        pltpu.VMEM((_CH, _D), jnp.float32),
        pltpu.VMEM((_CH, _D), jnp.float32),
        pltpu.VMEM((_CH, _D), jnp.float32),
        pltpu.VMEM((_DE, _D), jnp.float32),
        pltpu.VMEM_SHARED((_NPAD, _D), jnp.float32),
        pltpu.SemaphoreType.DMA,
        pltpu.SemaphoreType.DMA,
        pltpu.SemaphoreType.DMA,
        pltpu.SemaphoreType.DMA,
        pltpu.SemaphoreType.DMA,
        pltpu.SemaphoreType.DMA,
    ],
    )


def _edge_body(a_hbm, b_hbm, src_hbm, dst_hbm, ea_hbm, wme_hbm,
               z_hbm, out_hbm, src_a, dst_a, ea_a, src_b, dst_b, ea_b,
               arows_a, brows_a, arows_b, brows_b, wmev,
               aggsh, semi_a, semi_b, sga_a, sgb_a, sga_b, sgb_b):
    c = lax.axis_index("c")
    s = lax.axis_index("s")
    wid = s * _NC + c

    bufs = (
        (src_a, dst_a, ea_a, arows_a, brows_a, semi_a, sga_a, sgb_a),
        (src_b, dst_b, ea_b, arows_b, brows_b, semi_b, sga_b, sgb_b),
    )

    def issue_idx(ci, p):
        srcv, dstv, eav, _, _, semi, _, _ = bufs[p]
        base = pl.multiple_of(wid * _EPW + ci * _CH, 8)
        pltpu.async_copy(src_hbm.at[pl.ds(base, _CH)], srcv, semi)
        pltpu.async_copy(dst_hbm.at[pl.ds(base, _CH)], dstv, semi)
        pltpu.async_copy(ea_hbm.at[pl.ds(base * _DE, _CH * _DE)], eav, semi)

    def wait_idx(p):
        srcv, dstv, eav, _, _, semi, _, _ = bufs[p]
        pltpu.make_async_copy(src_hbm.at[pl.ds(0, _CH)], srcv, semi).wait()
        pltpu.make_async_copy(dst_hbm.at[pl.ds(0, _CH)], dstv, semi).wait()
        pltpu.make_async_copy(
            ea_hbm.at[pl.ds(0, _CH * _DE)], eav, semi).wait()

    def issue_gather(p):
        srcv, dstv, _, arows, brows, _, sga, sgb = bufs[p]
        pltpu.async_copy(a_hbm.at[srcv], arows, sga)
        pltpu.async_copy(b_hbm.at[dstv], brows, sgb)

    def wait_gather(p):
        srcv, dstv, _, arows, brows, _, sga, sgb = bufs[p]
        pltpu.make_async_copy(a_hbm.at[srcv], arows, sga).wait()
        pltpu.make_async_copy(b_hbm.at[dstv], brows, sgb).wait()

    def compute_scatter(p):
        _, dstv, eav, arows, brows, _, _, _ = bufs[p]

        jper = _D // 16 // _NPASS
        for ppass in range(_NPASS):
            jlo = ppass * jper

            wregs = [[wmev[k, pl.ds((jlo + j) * 16, 16)]
                      for j in range(jper)] for k in range(_DE)]

            @plsc.parallel_loop(0, _CH // 4, unroll=_UNROLL)
            def _egroup(g, wregs=wregs, jlo=jlo):
                w = eav[pl.ds(g * 16, 16)]
                for m in range(4):
                    e = g * 4 + m
                    ea0 = w[4 * m + 0]
                    ea1 = w[4 * m + 1]
                    ea2 = w[4 * m + 2]
                    ea3 = w[4 * m + 3]
                    for j in range(jper):
                        sl = pl.ds((jlo + j) * 16, 16)
                        ab = arows[e, sl] + brows[e, sl]
                        p01 = ea0 * wregs[0][j] + ea1 * wregs[1][j]
                        p23 = ea2 * wregs[2][j] + ea3 * wregs[3][j]
                        v = ab + (p01 + p23)
                        arows[e, sl] = jnp.maximum(v, 0.0)

        pltpu.sync_copy(arows, aggsh.at[dstv], add=True)

    issue_idx(0, 0)
    issue_idx(1, 1)

    pltpu.sync_copy(wme_hbm, wmev)

    pltpu.sync_copy(z_hbm, aggsh.at[pl.ds(s * _RPS, _RPS)])
    plsc.subcore_barrier()

    wait_idx(0)
    issue_gather(0)

    def _pair(k2, carry):
        c0 = k2 * 2
        wait_gather(0)
        wait_idx(1)
        issue_gather(1)
        compute_scatter(0)
        issue_idx(c0 + 2, 0)
        wait_gather(1)
        wait_idx(0)
        issue_gather(0)
        compute_scatter(1)

        @pl.when(c0 + 3 < _NCHUNK)
        def _():
            issue_idx(c0 + 3, 1)

        return carry

    lax.fori_loop(0, (_NCHUNK - 1) // 2, _pair, 0)

    wait_gather(0)
    compute_scatter(0)
    plsc.subcore_barrier()

    r0 = s * _RPS
    pltpu.sync_copy(aggsh.at[pl.ds(r0, _RPS)], out_hbm.at[c, pl.ds(r0, _RPS)])


def kernel(x, edge_index, edge_attr, W_enc1, b_enc1, W_enc2, b_enc2,
           Wm, bm, Wu, bu, W_dec1, b_dec1, W_dec2, b_dec2):
    src = edge_index[0]
    dst = edge_index[1]
    ea_flat = edge_attr.reshape(_E * _DE)
    zeros_blk = jnp.zeros((_RPS, _D), jnp.float32)
    b_enc1r = b_enc1.reshape(1, _D)
    b_enc2r = b_enc2.reshape(1, _D)
    b_dec1r = b_dec1.reshape(1, _D)
    b_dec2r = b_dec2.reshape(1, _OUT)

    h, a_proj, b_proj = _enc_call(
        x, W_enc1, b_enc1r, W_enc2, b_enc2r, Wm[0, :_D], Wm[0, _D:2 * _D],
        bm[0].reshape(1, _D))

    preds = None
    for i in range(_L):
        parts = _build_edge_pass()(a_proj, b_proj, src, dst, ea_flat,
                                   Wm[i, 2 * _D:], zeros_blk)
        p0 = parts[0, :_N]
        p1 = parts[1, :_N]
        wua = Wu[i, :_D]
        wub = Wu[i, _D:]
        bur = bu[i].reshape(1, _D)
        if i < _L - 1:
            h, a_proj, b_proj = _upd_call(
                h, p0, p1, wua, wub, bur,
                Wm[i + 1, :_D], Wm[i + 1, _D:2 * _D],
                bm[i + 1].reshape(1, _D))
        else:
            (preds,) = _fin_call(
                h, p0, p1, wua, wub, bur,
                W_dec1, b_dec1r, W_dec2, b_dec2r)
    return preds
